# baseline (speedup 1.0000x reference)
import functools

import jax
import jax.numpy as jnp
from jax import lax
from jax.experimental import pallas as pl
from jax.experimental.pallas import tpu as pltpu
from jax.experimental.pallas import tpu_sc as plsc

B, H, N = 32, 8, 100000
L = 16
NC, NS = 2, 16
TB = 128
NT = N // TB
WC = 31 * TB
NCH = NT // 31
REM = (NT - NCH * 31) * TB
TAILC = N - NT * TB
GTAIL = (NT * TB) // L
RING = 4


@functools.partial(
    pl.kernel,
    mesh=plsc.VectorSubcoreMesh(core_axis_name="c", subcore_axis_name="s"),
    out_type=jax.ShapeDtypeStruct((B * L,), jnp.float32),
    scratch_types=[
        pltpu.VMEM((RING, H, WC), jnp.float32),
        pltpu.VMEM((H, TB), jnp.float32),
        pltpu.VMEM((L,), jnp.float32),
        pltpu.SemaphoreType.DMA,
        pltpu.SemaphoreType.DMA,
        pltpu.SemaphoreType.DMA,
        pltpu.SemaphoreType.DMA,
        pltpu.SemaphoreType.DMA,
    ],
)
def _argmax_sc(x_hbm, xt_hbm, out_hbm, buf, tbuf, res,
               sem0, sem1, sem2, sem3, semt):
    c = lax.axis_index("c")
    s = lax.axis_index("s")
    b = s * NC + c
    sems = (sem0, sem1, sem2, sem3)
    iota = lax.iota(jnp.int32, L)

    widths = [WC] * NCH + [REM]
    starts = [i * WC for i in range(NCH + 1)]
    nchunks = NCH + 1

    def start(ci):
        slot = ci % RING
        w = widths[ci]
        return pltpu.async_copy(
            x_hbm.at[b, :, pl.ds(starts[ci], w)],
            buf.at[slot, :, pl.ds(0, w)],
            sems[slot])

    tcp = pltpu.async_copy(xt_hbm.at[b], tbuf, semt)
    cps = [start(ci) for ci in range(min(RING, nchunks))]

    ms = [jnp.full((L,), -jnp.inf, dtype=jnp.float32) for _ in range(H)]
    aas = [jnp.zeros((L,), dtype=jnp.int32) for _ in range(H)]

    def scan_groups(bufref, gbase, ngroups, ms, aas):
        def body(g, carry):
            mm = list(carry[:H])
            aa = list(carry[H:])
            col = g * L
            gv = jnp.broadcast_to(gbase + g, (L,))
            for r in range(H):
                v = bufref[r, pl.ds(col, L)]
                mm[r] = jnp.maximum(v, mm[r])
                aa[r] = aa[r]
            del gv
            return tuple(mm) + tuple(aa)

        carry = lax.fori_loop(0, ngroups, body, tuple(ms) + tuple(aas))
        return list(carry[:H]), list(carry[H:])

    for ci in range(nchunks):
        slot = ci % RING
        cps[slot].wait()
        ms, aas = scan_groups(
            buf.at[slot], starts[ci] // L, widths[ci] // L, ms, aas)
        if ci + RING < nchunks:
            cps[slot] = start(ci + RING)

    tcp.wait()
    ms, aas = scan_groups(tbuf, GTAIL, TB // L, ms, aas)

    resv = jnp.zeros((L,), dtype=jnp.float32)
    for r in range(H):
        rm = ms[r]
        ra = (aas[r] << 4) + iota
        for sh in (8, 4, 2, 1):
            perm = iota ^ sh
            mo = rm.at[perm].get(mode="promise_in_bounds")
            ao = ra.at[perm].get(mode="promise_in_bounds")
            better = (mo > rm) | ((mo == rm) & (ao < ra))
            rm = jnp.where(better, mo, rm)
            ra = jnp.where(better, ao, ra)
        val = ra.astype(jnp.float32) / jnp.float32(N)
        resv = jnp.where(iota == r, val, resv)

    res[...] = resv
    oout = pl.multiple_of(b * L, 8)
    pltpu.sync_copy(res, out_hbm.at[pl.ds(oout, L)])


def kernel(x):
    tail = lax.slice(x, (0, 0, NT * TB), (B, H, N))
    xt = jnp.pad(tail, ((0, 0), (0, 0), (0, TB - TAILC)),
                 constant_values=-jnp.inf)
    out = _argmax_sc(x, xt)
    return out.reshape(B, L)[:, :H]


# speedup vs baseline: 1.0101x; 1.0101x over previous
import functools

import jax
import jax.numpy as jnp
from jax import lax
from jax.experimental import pallas as pl
from jax.experimental.pallas import tpu as pltpu
from jax.experimental.pallas import tpu_sc as plsc

B, H, N = 32, 8, 100000
L = 16
NC, NS = 2, 16
TB = 128
NT = N // TB
CTW = 20
WC = CTW * TB
NCH = NT // CTW
REM = (NT - NCH * CTW) * TB
TAILC = N - NT * TB
GTAIL = (NT * TB) // L
RING = 6


@functools.partial(
    pl.kernel,
    mesh=plsc.VectorSubcoreMesh(core_axis_name="c", subcore_axis_name="s"),
    out_type=jax.ShapeDtypeStruct((B * L,), jnp.float32),
    scratch_types=[
        pltpu.VMEM((RING, H, WC), jnp.float32),
        pltpu.VMEM((H, TB), jnp.float32),
        pltpu.VMEM((L,), jnp.float32),
        pltpu.SemaphoreType.DMA,
        pltpu.SemaphoreType.DMA,
        pltpu.SemaphoreType.DMA,
        pltpu.SemaphoreType.DMA,
        pltpu.SemaphoreType.DMA,
        pltpu.SemaphoreType.DMA,
        pltpu.SemaphoreType.DMA,
    ],
)
def _argmax_sc(x_hbm, xt_hbm, out_hbm, buf, tbuf, res,
               sem0, sem1, sem2, sem3, sem4, sem5, semt):
    c = lax.axis_index("c")
    s = lax.axis_index("s")
    b = s * NC + c
    sems = (sem0, sem1, sem2, sem3, sem4, sem5)
    iota = lax.iota(jnp.int32, L)

    widths = [WC] * NCH + [REM]
    starts = [i * WC for i in range(NCH + 1)]
    nchunks = NCH + 1

    def start(ci):
        slot = ci % RING
        w = widths[ci]
        return pltpu.async_copy(
            x_hbm.at[b, :, pl.ds(starts[ci], w)],
            buf.at[slot, :, pl.ds(0, w)],
            sems[slot])

    tcp = pltpu.async_copy(xt_hbm.at[b], tbuf, semt)
    cps = [start(ci) for ci in range(min(RING, nchunks))]

    ms = [jnp.full((L,), -jnp.inf, dtype=jnp.float32) for _ in range(H)]
    aas = [jnp.zeros((L,), dtype=jnp.int32) for _ in range(H)]

    def scan_groups(bufref, gbase, ngroups, ms, aas):
        def body(g, carry):
            mm = list(carry[:H])
            aa = list(carry[H:])
            col = g * L
            gv = jnp.broadcast_to(gbase + g, (L,))
            for r in range(H):
                v = bufref[r, pl.ds(col, L)]
                gt = v > mm[r]
                mm[r] = jnp.where(gt, v, mm[r])
                aa[r] = jnp.where(gt, gv, aa[r])
            return tuple(mm) + tuple(aa)

        carry = lax.fori_loop(0, ngroups, body, tuple(ms) + tuple(aas))
        return list(carry[:H]), list(carry[H:])

    for ci in range(nchunks):
        slot = ci % RING
        cps[slot].wait()
        ms, aas = scan_groups(
            buf.at[slot], starts[ci] // L, widths[ci] // L, ms, aas)
        if ci + RING < nchunks:
            cps[slot] = start(ci + RING)

    tcp.wait()
    ms, aas = scan_groups(tbuf, GTAIL, TB // L, ms, aas)

    resv = jnp.zeros((L,), dtype=jnp.float32)
    for r in range(H):
        rm = ms[r]
        ra = (aas[r] << 4) + iota
        for sh in (8, 4, 2, 1):
            perm = iota ^ sh
            mo = rm.at[perm].get(mode="promise_in_bounds")
            ao = ra.at[perm].get(mode="promise_in_bounds")
            better = (mo > rm) | ((mo == rm) & (ao < ra))
            rm = jnp.where(better, mo, rm)
            ra = jnp.where(better, ao, ra)
        val = ra.astype(jnp.float32) / jnp.float32(N)
        resv = jnp.where(iota == r, val, resv)

    res[...] = resv
    oout = pl.multiple_of(b * L, 8)
    pltpu.sync_copy(res, out_hbm.at[pl.ds(oout, L)])


def kernel(x):
    tail = lax.slice(x, (0, 0, NT * TB), (B, H, N))
    xt = jnp.pad(tail, ((0, 0), (0, 0), (0, TB - TAILC)),
                 constant_values=-jnp.inf)
    out = _argmax_sc(x, xt)
    return out.reshape(B, L)[:, :H]
